# baseline (speedup 1.0000x reference)
import jax
import jax.numpy as jnp
from jax import lax
from jax.experimental import pallas as pl
from jax.experimental.pallas import tpu as pltpu
from jax.experimental.pallas import tpu_sc as plsc

NUM_CORES = 2
NUM_SUBCORES = 16
LANES = 16
NUM_WORKERS = NUM_CORES * NUM_SUBCORES

B = 16384
NUM_FACTOR = 64
NUM_DIM = NUM_FACTOR // 4
B_PER_W = B // NUM_WORKERS
GATHER_CHUNK = 128
NUM_CHUNKS = B_PER_W // GATHER_CHUNK
GROUPS = B_PER_W // LANES


def _sc_kernel(dow_hbm, time_hbm, month_hbm, day_hbm, dest_hbm,
               wdow_hbm, wtime_hbm, wmonth_hbm, wday_hbm, witem_hbm,
               out_hbm,
               dest_v, rows_v, dow_v, time_v, month_v, day_v,
               tdow_v, ttime_v, tmonth_v, tday_v, out_v, sem):
  wid = lax.axis_index("s") * NUM_CORES + lax.axis_index("c")
  base = wid * B_PER_W

  pltpu.sync_copy(dest_hbm.at[wid], dest_v)
  pltpu.sync_copy(dow_hbm.at[pl.ds(base, B_PER_W)], dow_v)
  pltpu.sync_copy(time_hbm.at[pl.ds(base, B_PER_W)], time_v)
  pltpu.sync_copy(month_hbm.at[pl.ds(base, B_PER_W)], month_v)
  pltpu.sync_copy(day_hbm.at[pl.ds(base, B_PER_W)], day_v)

  pltpu.sync_copy(wdow_hbm, tdow_v)
  pltpu.sync_copy(wtime_hbm, ttime_v)
  pltpu.sync_copy(wmonth_hbm, tmonth_v)
  pltpu.sync_copy(wday_hbm, tday_v)

  handles = [
      pltpu.async_copy(
          witem_hbm.at[dest_v.at[j]],
          rows_v.at[pl.ds(j * GATHER_CHUNK, GATHER_CHUNK)],
          sem,
      )
      for j in range(NUM_CHUNKS)
  ]
  for h in handles:
    h.wait()
  lane_iota = lax.iota(jnp.int32, LANES)
  tables = (tdow_v, ttime_v, tmonth_v, tday_v)
  idx_refs = (dow_v, time_v, month_v, day_v)

  def body(g, carry):
    off = g * LANES
    bvec = lane_iota + off
    acc = jnp.zeros((LANES,), jnp.float32)
    for seg in range(4):
      seg_idx = idx_refs[seg][pl.ds(off, LANES)]
      tab = tables[seg]
      for dd in range(NUM_DIM):
        col = jnp.full((LANES,), dd, jnp.int32)
        u = plsc.load_gather(tab, [seg_idx, col])
        icol = jnp.full((LANES,), seg * NUM_DIM + dd, jnp.int32)
        it = plsc.load_gather(rows_v, [bvec, icol])
        acc = acc + u * it
    out_v[pl.ds(off, LANES)] = acc
    return carry

  lax.fori_loop(0, GROUPS, body, 0)
  pltpu.sync_copy(out_v, out_hbm.at[pl.ds(base, B_PER_W)])


@jax.jit
def kernel(dayofweek, time, month, day, destination,
           W_dow, W_time, W_month, W_day, W_item):
  dow = dayofweek.astype(jnp.int32)
  tim = time.astype(jnp.int32)
  mon = month.astype(jnp.int32)
  dayi = day.astype(jnp.int32)
  dest = destination.astype(jnp.int32).reshape(NUM_WORKERS, NUM_CHUNKS,
                                               GATHER_CHUNK)

  mesh = plsc.VectorSubcoreMesh(
      core_axis_name="c", subcore_axis_name="s",
      num_cores=NUM_CORES, num_subcores=NUM_SUBCORES)

  run = pl.kernel(
      _sc_kernel,
      out_type=jax.ShapeDtypeStruct((B,), jnp.float32),
      mesh=mesh,
      scratch_types=[
          pltpu.VMEM((NUM_CHUNKS, GATHER_CHUNK), jnp.int32),
          pltpu.VMEM((B_PER_W, NUM_FACTOR), jnp.float32),
          pltpu.VMEM((B_PER_W,), jnp.int32),
          pltpu.VMEM((B_PER_W,), jnp.int32),
          pltpu.VMEM((B_PER_W,), jnp.int32),
          pltpu.VMEM((B_PER_W,), jnp.int32),
          pltpu.VMEM(W_dow.shape, jnp.float32),
          pltpu.VMEM(W_time.shape, jnp.float32),
          pltpu.VMEM(W_month.shape, jnp.float32),
          pltpu.VMEM(W_day.shape, jnp.float32),
          pltpu.VMEM((B_PER_W,), jnp.float32),
          pltpu.SemaphoreType.DMA,
      ],
      compiler_params=pltpu.CompilerParams(
          needs_layout_passes=False, use_tc_tiling_on_sc=False),
  )
  return run(dow, tim, mon, dayi, dest,
             W_dow, W_time, W_month, W_day, W_item)


# speedup vs baseline: 1.6145x; 1.6145x over previous
import jax
import jax.numpy as jnp
from jax import lax
from jax.experimental import pallas as pl
from jax.experimental.pallas import tpu as pltpu
from jax.experimental.pallas import tpu_sc as plsc

NUM_CORES = 2
NUM_SUBCORES = 16
LANES = 16
NUM_WORKERS = NUM_CORES * NUM_SUBCORES

B = 16384
NUM_FACTOR = 64
NUM_DIM = NUM_FACTOR // 4
B_PER_W = B // NUM_WORKERS
FLIGHT = 64
NUM_WAVES = B_PER_W // FLIGHT
GROUPS = B_PER_W // LANES


def _sc_kernel(dow_hbm, time_hbm, month_hbm, day_hbm, dest_hbm,
               wdow_hbm, wtime_hbm, wmonth_hbm, wday_hbm, witem_hbm,
               out_hbm,
               dest_v, rows_v, dow_v, time_v, month_v, day_v,
               tdow_v, ttime_v, tmonth_v, tday_v, out_v, sem):
  wid = lax.axis_index("s") * NUM_CORES + lax.axis_index("c")
  base = wid * B_PER_W

  pltpu.sync_copy(dest_hbm.at[pl.ds(base, B_PER_W)], dest_v)
  pltpu.sync_copy(dow_hbm.at[pl.ds(base, B_PER_W)], dow_v)
  pltpu.sync_copy(time_hbm.at[pl.ds(base, B_PER_W)], time_v)
  pltpu.sync_copy(month_hbm.at[pl.ds(base, B_PER_W)], month_v)
  pltpu.sync_copy(day_hbm.at[pl.ds(base, B_PER_W)], day_v)

  pltpu.sync_copy(wdow_hbm, tdow_v)
  pltpu.sync_copy(wtime_hbm, ttime_v)
  pltpu.sync_copy(wmonth_hbm, tmonth_v)
  pltpu.sync_copy(wday_hbm, tday_v)

  def dma_wave(w, carry):
    woff = w * FLIGHT
    handles = []
    for q in range(FLIGHT // LANES):
      ivec = dest_v[pl.ds(woff + q * LANES, LANES)]
      for u in range(LANES):
        i = woff + q * LANES + u
        handles.append(
            pltpu.async_copy(witem_hbm.at[pl.ds(ivec[u], 1)],
                             rows_v.at[pl.ds(i, 1)], sem))
    for h in handles:
      h.wait()
    return carry

  lax.fori_loop(0, NUM_WAVES, dma_wave, 0)

  lane_iota = lax.iota(jnp.int32, LANES)
  tables = (tdow_v, ttime_v, tmonth_v, tday_v)
  idx_refs = (dow_v, time_v, month_v, day_v)

  def body(g, carry):
    off = g * LANES
    bvec = lane_iota + off
    acc = jnp.zeros((LANES,), jnp.float32)
    for seg in range(4):
      seg_idx = idx_refs[seg][pl.ds(off, LANES)]
      tab = tables[seg]
      for dd in range(NUM_DIM):
        col = jnp.full((LANES,), dd, jnp.int32)
        u = plsc.load_gather(tab, [seg_idx, col])
        icol = jnp.full((LANES,), seg * NUM_DIM + dd, jnp.int32)
        it = plsc.load_gather(rows_v, [bvec, icol])
        acc = acc + u * it
    out_v[pl.ds(off, LANES)] = acc
    return carry

  lax.fori_loop(0, GROUPS, body, 0)
  pltpu.sync_copy(out_v, out_hbm.at[pl.ds(base, B_PER_W)])


@jax.jit
def kernel(dayofweek, time, month, day, destination,
           W_dow, W_time, W_month, W_day, W_item):
  dow = dayofweek.astype(jnp.int32)
  tim = time.astype(jnp.int32)
  mon = month.astype(jnp.int32)
  dayi = day.astype(jnp.int32)
  dest = destination.astype(jnp.int32)

  mesh = plsc.VectorSubcoreMesh(
      core_axis_name="c", subcore_axis_name="s",
      num_cores=NUM_CORES, num_subcores=NUM_SUBCORES)

  run = pl.kernel(
      _sc_kernel,
      out_type=jax.ShapeDtypeStruct((B,), jnp.float32),
      mesh=mesh,
      scratch_types=[
          pltpu.VMEM((B_PER_W,), jnp.int32),
          pltpu.VMEM((B_PER_W, NUM_FACTOR), jnp.float32),
          pltpu.VMEM((B_PER_W,), jnp.int32),
          pltpu.VMEM((B_PER_W,), jnp.int32),
          pltpu.VMEM((B_PER_W,), jnp.int32),
          pltpu.VMEM((B_PER_W,), jnp.int32),
          pltpu.VMEM(W_dow.shape, jnp.float32),
          pltpu.VMEM(W_time.shape, jnp.float32),
          pltpu.VMEM(W_month.shape, jnp.float32),
          pltpu.VMEM(W_day.shape, jnp.float32),
          pltpu.VMEM((B_PER_W,), jnp.float32),
          pltpu.SemaphoreType.DMA,
      ],
      compiler_params=pltpu.CompilerParams(needs_layout_passes=False),
  )
  return run(dow, tim, mon, dayi, dest,
             W_dow, W_time, W_month, W_day, W_item)
